# baseline (speedup 1.0000x reference)
import functools

import jax
import jax.numpy as jnp
from jax import lax
from jax.experimental import pallas as pl
from jax.experimental.pallas import tpu as pltpu
from jax.experimental.pallas import tpu_sc as plsc

N = 10000
E = 320000
D = 128

NC = 2
NS = 16
NW = NC * NS
CH = 128
K = 80
KH = K // 2
E_PAD = NW * K * CH
N_PAD = 10240
BLK_PER_SUB = (N_PAD // CH) // NS
PAD_ROW = N
DEG_W = 128

_mesh = plsc.VectorSubcoreMesh(core_axis_name="c", subcore_axis_name="s")
_f32 = jnp.float32


def _deg_body(w, dst_hbm, out_hbm, dstv, buf, accd):
    c = lax.axis_index("c")
    s = lax.axis_index("s")
    wid = s * NC + c
    pltpu.sync_copy(dst_hbm.at[wid], dstv)

    zero = jnp.zeros((16,), _f32)
    one = jnp.ones((16,), _f32)

    @pl.loop(0, CH)
    def _(r):
        @pl.loop(0, w, step=16)
        def _(cc):
            buf[r, pl.ds(cc, 16)] = zero

    @pl.loop(0, BLK_PER_SUB)
    def _(b):
        off = (s * BLK_PER_SUB + b) * CH
        pltpu.sync_copy(buf, accd.at[pl.ds(off, CH)])

    @pl.loop(0, CH)
    def _(r):
        buf[r, pl.ds(0, 16)] = one

    plsc.subcore_barrier()

    @pl.loop(0, K)
    def _(j):
        pltpu.sync_copy(buf, accd.at[dstv.at[j]], add=True)

    plsc.subcore_barrier()

    @pl.loop(0, BLK_PER_SUB)
    def _(b):
        off = (s * BLK_PER_SUB + b) * CH
        pltpu.sync_copy(accd.at[pl.ds(off, CH)], out_hbm.at[c].at[pl.ds(off, CH)])


def _make_deg_call(w):
    return pl.kernel(
        functools.partial(_deg_body, w),
        out_type=jax.ShapeDtypeStruct((NC, N_PAD, w), _f32),
        mesh=_mesh,
        scratch_types=[
            pltpu.VMEM((K, CH), jnp.int32),
            pltpu.VMEM((CH, w), _f32),
            pltpu.VMEM_SHARED((N_PAD, w), _f32),
        ],
    )


_deg_call = _make_deg_call(DEG_W)


K0 = 106
K0H = K0 // 2
K1 = 2 * K - K0
K1H = K1 // 2


def _seg_body(h_hbm, src0_hbm, dst0_hbm, src1_hbm, dst1_hbm, out_hbm,
              srcv, dstv, rows, acc):
    c = lax.axis_index("c")
    s = lax.axis_index("s")

    zero = jnp.zeros((16,), _f32)

    @pl.loop(0, CH)
    def _(r):
        @pl.loop(0, D, step=16)
        def _(cc):
            rows[r, pl.ds(cc, 16)] = zero

    @pl.loop(0, BLK_PER_SUB)
    def _(b):
        off = (s * BLK_PER_SUB + b) * CH
        pltpu.sync_copy(rows, acc.at[pl.ds(off, CH)])

    plsc.subcore_barrier()

    @pl.when(c == 0)
    def _():
        @pl.loop(0, 2)
        def _(hf):
            pltpu.sync_copy(src0_hbm.at[s].at[hf], srcv)
            pltpu.sync_copy(dst0_hbm.at[s].at[hf], dstv)

            @pl.loop(0, K0H)
            def _(j):
                pltpu.sync_copy(h_hbm.at[srcv.at[j]], rows)
                pltpu.sync_copy(rows, acc.at[dstv.at[j]], add=True)

    @pl.when(c == 1)
    def _():
        @pl.loop(0, 2)
        def _(hf):
            pltpu.sync_copy(src1_hbm.at[s].at[hf], srcv.at[pl.ds(0, K1H)])
            pltpu.sync_copy(dst1_hbm.at[s].at[hf], dstv.at[pl.ds(0, K1H)])

            @pl.loop(0, K1H)
            def _(j):
                pltpu.sync_copy(h_hbm.at[srcv.at[j]], rows)
                pltpu.sync_copy(rows, acc.at[dstv.at[j]], add=True)

    plsc.subcore_barrier()

    @pl.loop(0, BLK_PER_SUB)
    def _(b):
        off = (s * BLK_PER_SUB + b) * CH
        pltpu.sync_copy(acc.at[pl.ds(off, CH)], out_hbm.at[c].at[pl.ds(off, CH)])


_seg_call = pl.kernel(
    _seg_body,
    out_type=jax.ShapeDtypeStruct((NC, N_PAD, D), _f32),
    mesh=_mesh,
    scratch_types=[
        pltpu.VMEM((K0H, CH), jnp.int32),
        pltpu.VMEM((K0H, CH), jnp.int32),
        pltpu.VMEM((CH, D), _f32),
        pltpu.VMEM_SHARED((N_PAD, D), _f32),
    ],
)


B = 256


def _dinv_block(degp_ref):
    deg = degp_ref[0] + degp_ref[1] + 1.0
    return lax.rsqrt(deg)[:, 0:1]


def _row_mask(i):
    row = lax.broadcasted_iota(jnp.int32, (B, 1), 0) + i * B
    return row < N


def _tc1_body(x_ref, w_ref, degp_ref, out_ref):
    dinv = _dinv_block(degp_ref)
    h = jnp.dot(x_ref[...], w_ref[...], preferred_element_type=_f32)
    out_ref[...] = jnp.where(_row_mask(pl.program_id(0)), h * dinv, 0.0)


_tc1_call = pl.pallas_call(
    _tc1_body,
    grid=(N_PAD // B,),
    in_specs=[
        pl.BlockSpec((B, D), lambda i: (i, 0)),
        pl.BlockSpec((D, D), lambda i: (0, 0)),
        pl.BlockSpec((NC, B, DEG_W), lambda i: (0, i, 0)),
    ],
    out_specs=pl.BlockSpec((B, D), lambda i: (i, 0)),
    out_shape=jax.ShapeDtypeStruct((N_PAD, D), _f32),
)


def _tc2_body(s1_ref, h1_ref, degp_ref, w_ref, b1_ref, out_ref):
    dinv = _dinv_block(degp_ref)
    t = (s1_ref[0] + s1_ref[1] + h1_ref[...]) * dinv + b1_ref[...]
    z = jnp.where(t >= 0, t, 0.01 * t)
    h2 = jnp.dot(z, w_ref[...], preferred_element_type=_f32) * dinv
    out_ref[...] = jnp.where(_row_mask(pl.program_id(0)), h2, 0.0)


_tc2_call = pl.pallas_call(
    _tc2_body,
    grid=(N_PAD // B,),
    in_specs=[
        pl.BlockSpec((NC, B, D), lambda i: (0, i, 0)),
        pl.BlockSpec((B, D), lambda i: (i, 0)),
        pl.BlockSpec((NC, B, DEG_W), lambda i: (0, i, 0)),
        pl.BlockSpec((D, D), lambda i: (0, 0)),
        pl.BlockSpec((1, D), lambda i: (0, 0)),
    ],
    out_specs=pl.BlockSpec((B, D), lambda i: (i, 0)),
    out_shape=jax.ShapeDtypeStruct((N_PAD, D), _f32),
)


def _tc3_body(s2_ref, h2_ref, degp_ref, b2_ref, out_ref):
    dinv = _dinv_block(degp_ref)
    out_ref[...] = (s2_ref[0] + s2_ref[1] + h2_ref[...]) * dinv + b2_ref[...]


_tc3_call = pl.pallas_call(
    _tc3_body,
    grid=(N_PAD // B,),
    in_specs=[
        pl.BlockSpec((NC, B, D), lambda i: (0, i, 0)),
        pl.BlockSpec((B, D), lambda i: (i, 0)),
        pl.BlockSpec((NC, B, DEG_W), lambda i: (0, i, 0)),
        pl.BlockSpec((1, D), lambda i: (0, 0)),
    ],
    out_specs=pl.BlockSpec((B, D), lambda i: (i, 0)),
    out_shape=jax.ShapeDtypeStruct((N_PAD, D), _f32),
)


@jax.jit
def kernel(x, edge_index, W1, b1, W2, b2):
    src = edge_index[0].astype(jnp.int32)
    dst = edge_index[1].astype(jnp.int32)
    pad = jnp.full((E_PAD - E,), PAD_ROW, jnp.int32)
    src_f = jnp.concatenate([src, pad])
    dst_f = jnp.concatenate([dst, pad])
    dst_p = dst_f.reshape(NW, K, CH)
    x_pad = jnp.concatenate([x, jnp.zeros((N_PAD - N, D), x.dtype)])

    e0 = NS * K0 * CH
    src0 = src_f[:e0].reshape(NS, 2, K0H, CH)
    dst0 = dst_f[:e0].reshape(NS, 2, K0H, CH)
    src1 = src_f[e0:].reshape(NS, 2, K1H, CH)
    dst1 = dst_f[e0:].reshape(NS, 2, K1H, CH)

    degp = _deg_call(dst_p)
    h1p = _tc1_call(x_pad, W1, degp)
    s1p = _seg_call(h1p, src0, dst0, src1, dst1)
    h2p = _tc2_call(s1p, h1p, degp, W2, b1.reshape(1, D))
    s2p = _seg_call(h2p, src0, dst0, src1, dst1)
    out = _tc3_call(s2p, h2p, degp, b2.reshape(1, D))
    return out[:N]


# speedup vs baseline: 1.0014x; 1.0014x over previous
import functools

import jax
import jax.numpy as jnp
from jax import lax
from jax.experimental import pallas as pl
from jax.experimental.pallas import tpu as pltpu
from jax.experimental.pallas import tpu_sc as plsc

N = 10000
E = 320000
D = 128

NC = 2
NS = 16
NW = NC * NS
CH = 128
K = 80
KH = K // 2
E_PAD = NW * K * CH
N_PAD = 10240
BLK_PER_SUB = (N_PAD // CH) // NS
PAD_ROW = N
DEG_W = 128

_mesh = plsc.VectorSubcoreMesh(core_axis_name="c", subcore_axis_name="s")
_f32 = jnp.float32


def _deg_body(w, dst_hbm, out_hbm, dstv, buf, accd):
    c = lax.axis_index("c")
    s = lax.axis_index("s")
    wid = s * NC + c
    pltpu.sync_copy(dst_hbm.at[wid], dstv)

    zero = jnp.zeros((16,), _f32)
    one = jnp.ones((16,), _f32)

    @pl.loop(0, CH)
    def _(r):
        @pl.loop(0, w, step=16)
        def _(cc):
            buf[r, pl.ds(cc, 16)] = zero

    @pl.loop(0, BLK_PER_SUB)
    def _(b):
        off = (s * BLK_PER_SUB + b) * CH
        pltpu.sync_copy(buf, accd.at[pl.ds(off, CH)])

    @pl.loop(0, CH)
    def _(r):
        buf[r, pl.ds(0, 16)] = one

    plsc.subcore_barrier()

    @pl.loop(0, K)
    def _(j):
        pltpu.sync_copy(buf, accd.at[dstv.at[j]], add=True)

    plsc.subcore_barrier()

    @pl.loop(0, BLK_PER_SUB)
    def _(b):
        off = (s * BLK_PER_SUB + b) * CH
        pltpu.sync_copy(accd.at[pl.ds(off, CH)], out_hbm.at[c].at[pl.ds(off, CH)])


def _make_deg_call(w):
    return pl.kernel(
        functools.partial(_deg_body, w),
        out_type=jax.ShapeDtypeStruct((NC, N_PAD, w), _f32),
        mesh=_mesh,
        scratch_types=[
            pltpu.VMEM((K, CH), jnp.int32),
            pltpu.VMEM((CH, w), _f32),
            pltpu.VMEM_SHARED((N_PAD, w), _f32),
        ],
    )


_deg_call = _make_deg_call(DEG_W)


K0 = 106
K0H = K0 // 2
K1 = 2 * K - K0
K1H = K1 // 2


def _seg_body(h_hbm, src0_hbm, dst0_hbm, src1_hbm, dst1_hbm, out_hbm,
              srcv, dstv, srcv1, dstv1, rows, acc):
    c = lax.axis_index("c")
    s = lax.axis_index("s")

    zero = jnp.zeros((16,), _f32)

    @pl.loop(0, CH)
    def _(r):
        @pl.loop(0, D, step=16)
        def _(cc):
            rows[r, pl.ds(cc, 16)] = zero

    @pl.loop(0, BLK_PER_SUB)
    def _(b):
        off = (s * BLK_PER_SUB + b) * CH
        pltpu.sync_copy(rows, acc.at[pl.ds(off, CH)])

    plsc.subcore_barrier()

    @pl.when(c == 0)
    def _():
        @pl.loop(0, 2)
        def _(hf):
            pltpu.sync_copy(src0_hbm.at[s].at[hf], srcv)
            pltpu.sync_copy(dst0_hbm.at[s].at[hf], dstv)

            @pl.loop(0, K0H)
            def _(j):
                pltpu.sync_copy(h_hbm.at[srcv.at[j]], rows)
                pltpu.sync_copy(rows, acc.at[dstv.at[j]], add=True)

    @pl.when(c == 1)
    def _():
        pltpu.sync_copy(src1_hbm.at[s], srcv1)
        pltpu.sync_copy(dst1_hbm.at[s], dstv1)

        @pl.loop(0, K1)
        def _(j):
            pltpu.sync_copy(h_hbm.at[srcv1.at[j]], rows)
            pltpu.sync_copy(rows, acc.at[dstv1.at[j]], add=True)

    plsc.subcore_barrier()

    @pl.loop(0, BLK_PER_SUB)
    def _(b):
        off = (s * BLK_PER_SUB + b) * CH
        pltpu.sync_copy(acc.at[pl.ds(off, CH)], out_hbm.at[c].at[pl.ds(off, CH)])


_seg_call = pl.kernel(
    _seg_body,
    out_type=jax.ShapeDtypeStruct((NC, N_PAD, D), _f32),
    mesh=_mesh,
    scratch_types=[
        pltpu.VMEM((K0H, CH), jnp.int32),
        pltpu.VMEM((K0H, CH), jnp.int32),
        pltpu.VMEM((K1, CH), jnp.int32),
        pltpu.VMEM((K1, CH), jnp.int32),
        pltpu.VMEM((CH, D), _f32),
        pltpu.VMEM_SHARED((N_PAD, D), _f32),
    ],
)


B = 256


def _dinv_block(degp_ref):
    deg = degp_ref[0] + degp_ref[1] + 1.0
    return lax.rsqrt(deg)[:, 0:1]


def _row_mask(i):
    row = lax.broadcasted_iota(jnp.int32, (B, 1), 0) + i * B
    return row < N


def _tc1_body(x_ref, w_ref, degp_ref, out_ref):
    dinv = _dinv_block(degp_ref)
    h = jnp.dot(x_ref[...], w_ref[...], preferred_element_type=_f32)
    out_ref[...] = jnp.where(_row_mask(pl.program_id(0)), h * dinv, 0.0)


_tc1_call = pl.pallas_call(
    _tc1_body,
    grid=(N_PAD // B,),
    in_specs=[
        pl.BlockSpec((B, D), lambda i: (i, 0)),
        pl.BlockSpec((D, D), lambda i: (0, 0)),
        pl.BlockSpec((NC, B, DEG_W), lambda i: (0, i, 0)),
    ],
    out_specs=pl.BlockSpec((B, D), lambda i: (i, 0)),
    out_shape=jax.ShapeDtypeStruct((N_PAD, D), _f32),
)


def _tc2_body(s1_ref, h1_ref, degp_ref, w_ref, b1_ref, out_ref):
    dinv = _dinv_block(degp_ref)
    t = (s1_ref[0] + s1_ref[1] + h1_ref[...]) * dinv + b1_ref[...]
    z = jnp.where(t >= 0, t, 0.01 * t)
    h2 = jnp.dot(z, w_ref[...], preferred_element_type=_f32) * dinv
    out_ref[...] = jnp.where(_row_mask(pl.program_id(0)), h2, 0.0)


_tc2_call = pl.pallas_call(
    _tc2_body,
    grid=(N_PAD // B,),
    in_specs=[
        pl.BlockSpec((NC, B, D), lambda i: (0, i, 0)),
        pl.BlockSpec((B, D), lambda i: (i, 0)),
        pl.BlockSpec((NC, B, DEG_W), lambda i: (0, i, 0)),
        pl.BlockSpec((D, D), lambda i: (0, 0)),
        pl.BlockSpec((1, D), lambda i: (0, 0)),
    ],
    out_specs=pl.BlockSpec((B, D), lambda i: (i, 0)),
    out_shape=jax.ShapeDtypeStruct((N_PAD, D), _f32),
)


def _tc3_body(s2_ref, h2_ref, degp_ref, b2_ref, out_ref):
    dinv = _dinv_block(degp_ref)
    out_ref[...] = (s2_ref[0] + s2_ref[1] + h2_ref[...]) * dinv + b2_ref[...]


_tc3_call = pl.pallas_call(
    _tc3_body,
    grid=(N_PAD // B,),
    in_specs=[
        pl.BlockSpec((NC, B, D), lambda i: (0, i, 0)),
        pl.BlockSpec((B, D), lambda i: (i, 0)),
        pl.BlockSpec((NC, B, DEG_W), lambda i: (0, i, 0)),
        pl.BlockSpec((1, D), lambda i: (0, 0)),
    ],
    out_specs=pl.BlockSpec((B, D), lambda i: (i, 0)),
    out_shape=jax.ShapeDtypeStruct((N_PAD, D), _f32),
)


@jax.jit
def kernel(x, edge_index, W1, b1, W2, b2):
    src = edge_index[0].astype(jnp.int32)
    dst = edge_index[1].astype(jnp.int32)
    pad = jnp.full((E_PAD - E,), PAD_ROW, jnp.int32)
    src_f = jnp.concatenate([src, pad])
    dst_f = jnp.concatenate([dst, pad])
    dst_p = dst_f.reshape(NW, K, CH)
    x_pad = jnp.concatenate([x, jnp.zeros((N_PAD - N, D), x.dtype)])

    e0 = NS * K0 * CH
    src0 = src_f[:e0].reshape(NS, 2, K0H, CH)
    dst0 = dst_f[:e0].reshape(NS, 2, K0H, CH)
    src1 = src_f[e0:].reshape(NS, K1, CH)
    dst1 = dst_f[e0:].reshape(NS, K1, CH)

    degp = _deg_call(dst_p)
    h1p = _tc1_call(x_pad, W1, degp)
    s1p = _seg_call(h1p, src0, dst0, src1, dst1)
    h2p = _tc2_call(s1p, h1p, degp, W2, b1.reshape(1, D))
    s2p = _seg_call(h2p, src0, dst0, src1, dst1)
    out = _tc3_call(s2p, h2p, degp, b2.reshape(1, D))
    return out[:N]


# speedup vs baseline: 1.7367x; 1.7343x over previous
import functools

import jax
import jax.numpy as jnp
from jax import lax
from jax.experimental import pallas as pl
from jax.experimental.pallas import tpu as pltpu
from jax.experimental.pallas import tpu_sc as plsc

N = 10000
E = 320000
D = 128

NC = 2
NS = 16
NW = NC * NS
CH = 128
K = 80
KH = K // 2
E_PAD = NW * K * CH
N_PAD = 10240
BLK_PER_SUB = (N_PAD // CH) // NS
PAD_ROW = N
DEG_W = 128

_mesh = plsc.VectorSubcoreMesh(core_axis_name="c", subcore_axis_name="s")
_f32 = jnp.float32


def _deg_body(w, dst_hbm, out_hbm, dstv, buf, accd):
    c = lax.axis_index("c")
    s = lax.axis_index("s")
    wid = s * NC + c
    pltpu.sync_copy(dst_hbm.at[wid], dstv)

    zero = jnp.zeros((16,), _f32)
    one = jnp.ones((16,), _f32)

    @pl.loop(0, CH)
    def _(r):
        @pl.loop(0, w, step=16)
        def _(cc):
            buf[r, pl.ds(cc, 16)] = zero

    @pl.loop(0, BLK_PER_SUB)
    def _(b):
        off = (s * BLK_PER_SUB + b) * CH
        pltpu.sync_copy(buf, accd.at[pl.ds(off, CH)])

    @pl.loop(0, CH)
    def _(r):
        buf[r, pl.ds(0, 16)] = one

    plsc.subcore_barrier()

    @pl.loop(0, K)
    def _(j):
        pltpu.sync_copy(buf, accd.at[dstv.at[j]], add=True)

    plsc.subcore_barrier()

    @pl.loop(0, BLK_PER_SUB)
    def _(b):
        off = (s * BLK_PER_SUB + b) * CH
        pltpu.sync_copy(accd.at[pl.ds(off, CH)], out_hbm.at[c].at[pl.ds(off, CH)])


def _make_deg_call(w):
    return pl.kernel(
        functools.partial(_deg_body, w),
        out_type=jax.ShapeDtypeStruct((NC, N_PAD, w), _f32),
        mesh=_mesh,
        scratch_types=[
            pltpu.VMEM((K, CH), jnp.int32),
            pltpu.VMEM((CH, w), _f32),
            pltpu.VMEM_SHARED((N_PAD, w), _f32),
        ],
    )


_deg_call = _make_deg_call(DEG_W)


K0 = 106
K0H = K0 // 2
K1 = 2 * K - K0
K1H = K1 // 2


def _seg_body(h_hbm, src0_hbm, dst0_hbm, src1_hbm, dst1_hbm, out_hbm,
              srcv, dstv, srcv1, dstv1, rows, acc):
    c = lax.axis_index("c")
    s = lax.axis_index("s")

    zero = jnp.zeros((16,), _f32)

    @pl.loop(0, CH)
    def _(r):
        @pl.loop(0, D, step=16)
        def _(cc):
            rows[r, pl.ds(cc, 16)] = zero

    @pl.loop(0, BLK_PER_SUB)
    def _(b):
        off = (s * BLK_PER_SUB + b) * CH
        pltpu.sync_copy(rows, acc.at[pl.ds(off, CH)])

    plsc.subcore_barrier()

    @pl.when(c == 0)
    def _():
        @pl.loop(0, 2)
        def _(hf):
            pltpu.sync_copy(src0_hbm.at[s].at[hf], srcv)
            pltpu.sync_copy(dst0_hbm.at[s].at[hf], dstv)

            @pl.loop(0, K0H)
            def _(j):
                pltpu.sync_copy(h_hbm.at[srcv.at[j]], rows)
                pltpu.sync_copy(rows, acc.at[dstv.at[j]], add=True)

    @pl.when(c == 1)
    def _():
        pltpu.sync_copy(src1_hbm.at[s], srcv1)
        pltpu.sync_copy(dst1_hbm.at[s], dstv1)

        @pl.loop(0, K1)
        def _(j):
            pltpu.sync_copy(h_hbm.at[srcv1.at[j]], rows)
            pltpu.sync_copy(rows, acc.at[dstv1.at[j]], add=True)

    plsc.subcore_barrier()

    @pl.loop(0, BLK_PER_SUB)
    def _(b):
        off = (s * BLK_PER_SUB + b) * CH
        pltpu.sync_copy(acc.at[pl.ds(off, CH)], out_hbm.at[c].at[pl.ds(off, CH)])


_seg_call = pl.kernel(
    _seg_body,
    out_type=jax.ShapeDtypeStruct((NC, N_PAD, D), _f32),
    mesh=_mesh,
    scratch_types=[
        pltpu.VMEM((K0H, CH), jnp.int32),
        pltpu.VMEM((K0H, CH), jnp.int32),
        pltpu.VMEM((K1, CH), jnp.int32),
        pltpu.VMEM((K1, CH), jnp.int32),
        pltpu.VMEM((CH, D), _f32),
        pltpu.VMEM_SHARED((N_PAD, D), _f32),
    ],
)


B = 256


def _dinv_block(degp_ref):
    deg = degp_ref[0] + degp_ref[1] + 1.0
    return lax.rsqrt(deg)[:, 0:1]


def _row_mask(i):
    row = lax.broadcasted_iota(jnp.int32, (B, 1), 0) + i * B
    return row < N


def _tc1_body(x_ref, w_ref, degp_ref, out_ref):
    dinv = _dinv_block(degp_ref)
    h = jnp.dot(x_ref[...], w_ref[...], preferred_element_type=_f32)
    out_ref[...] = jnp.where(_row_mask(pl.program_id(0)), h * dinv, 0.0)


_tc1_call = pl.pallas_call(
    _tc1_body,
    grid=(N_PAD // B,),
    in_specs=[
        pl.BlockSpec((B, D), lambda i: (i, 0)),
        pl.BlockSpec((D, D), lambda i: (0, 0)),
        pl.BlockSpec((NC, B, DEG_W), lambda i: (0, i, 0)),
    ],
    out_specs=pl.BlockSpec((B, D), lambda i: (i, 0)),
    out_shape=jax.ShapeDtypeStruct((N_PAD, D), _f32),
)


def _tc2_body(s1_ref, h1_ref, degp_ref, w_ref, b1_ref, out_ref):
    dinv = _dinv_block(degp_ref)
    t = (s1_ref[0] + s1_ref[1] + h1_ref[...]) * dinv + b1_ref[...]
    z = jnp.where(t >= 0, t, 0.01 * t)
    h2 = jnp.dot(z, w_ref[...], preferred_element_type=_f32) * dinv
    out_ref[...] = jnp.where(_row_mask(pl.program_id(0)), h2, 0.0)


_tc2_call = pl.pallas_call(
    _tc2_body,
    grid=(N_PAD // B,),
    in_specs=[
        pl.BlockSpec((NC, B, D), lambda i: (0, i, 0)),
        pl.BlockSpec((B, D), lambda i: (i, 0)),
        pl.BlockSpec((NC, B, DEG_W), lambda i: (0, i, 0)),
        pl.BlockSpec((D, D), lambda i: (0, 0)),
        pl.BlockSpec((1, D), lambda i: (0, 0)),
    ],
    out_specs=pl.BlockSpec((B, D), lambda i: (i, 0)),
    out_shape=jax.ShapeDtypeStruct((N_PAD, D), _f32),
)


def _tc3_body(s2_ref, h2_ref, degp_ref, b2_ref, out_ref):
    dinv = _dinv_block(degp_ref)
    out_ref[...] = (s2_ref[0] + s2_ref[1] + h2_ref[...]) * dinv + b2_ref[...]


_tc3_call = pl.pallas_call(
    _tc3_body,
    grid=(N_PAD // B,),
    in_specs=[
        pl.BlockSpec((NC, B, D), lambda i: (0, i, 0)),
        pl.BlockSpec((B, D), lambda i: (i, 0)),
        pl.BlockSpec((NC, B, DEG_W), lambda i: (0, i, 0)),
        pl.BlockSpec((1, D), lambda i: (0, 0)),
    ],
    out_specs=pl.BlockSpec((B, D), lambda i: (i, 0)),
    out_shape=jax.ShapeDtypeStruct((N_PAD, D), _f32),
)


@jax.jit
def kernel(x, edge_index, W1, b1, W2, b2):
    src = edge_index[0].astype(jnp.int32)
    dst = edge_index[1].astype(jnp.int32)
    pad = PAD_ROW + (jnp.arange(E_PAD - E, dtype=jnp.int32) % (N_PAD - N))
    src_f = jnp.concatenate([src, pad])
    dst_f = jnp.concatenate([dst, pad])
    dst_p = dst_f.reshape(NW, K, CH)
    x_pad = jnp.concatenate([x, jnp.zeros((N_PAD - N, D), x.dtype)])

    e0 = NS * K0 * CH
    src0 = src_f[:e0].reshape(NS, 2, K0H, CH)
    dst0 = dst_f[:e0].reshape(NS, 2, K0H, CH)
    src1 = src_f[e0:].reshape(NS, K1, CH)
    dst1 = dst_f[e0:].reshape(NS, K1, CH)

    degp = _deg_call(dst_p)
    h1p = _tc1_call(x_pad, W1, degp)
    s1p = _seg_call(h1p, src0, dst0, src1, dst1)
    h2p = _tc2_call(s1p, h1p, degp, W2, b1.reshape(1, D))
    s2p = _seg_call(h2p, src0, dst0, src1, dst1)
    out = _tc3_call(s2p, h2p, degp, b2.reshape(1, D))
    return out[:N]


# speedup vs baseline: 2.4785x; 1.4271x over previous
import functools

import jax
import jax.numpy as jnp
from jax import lax
from jax.experimental import pallas as pl
from jax.experimental.pallas import tpu as pltpu
from jax.experimental.pallas import tpu_sc as plsc

N = 10000
E = 320000
D = 128

NC = 2
NS = 16
NW = NC * NS
CH = 128
K = 80
KH = K // 2
E_PAD = NW * K * CH
N_PAD = 10240
BLK_PER_SUB = (N_PAD // CH) // NS
PAD_ROW = N
DEG_W = 128

_mesh = plsc.VectorSubcoreMesh(core_axis_name="c", subcore_axis_name="s")
_f32 = jnp.float32


def _deg_body(w, dst_hbm, out_hbm, dstv, buf, accd):
    c = lax.axis_index("c")
    s = lax.axis_index("s")
    wid = s * NC + c
    pltpu.sync_copy(dst_hbm.at[wid], dstv)

    zero = jnp.zeros((16,), _f32)
    one = jnp.ones((16,), _f32)

    @pl.loop(0, CH)
    def _(r):
        @pl.loop(0, w, step=16)
        def _(cc):
            buf[r, pl.ds(cc, 16)] = zero

    @pl.loop(0, BLK_PER_SUB)
    def _(b):
        off = (s * BLK_PER_SUB + b) * CH
        pltpu.sync_copy(buf, accd.at[pl.ds(off, CH)])

    @pl.loop(0, CH)
    def _(r):
        buf[r, pl.ds(0, 16)] = one

    plsc.subcore_barrier()

    @pl.loop(0, K)
    def _(j):
        pltpu.sync_copy(buf, accd.at[dstv.at[j]], add=True)

    plsc.subcore_barrier()

    @pl.loop(0, BLK_PER_SUB)
    def _(b):
        off = (s * BLK_PER_SUB + b) * CH
        pltpu.sync_copy(accd.at[pl.ds(off, CH)], out_hbm.at[c].at[pl.ds(off, CH)])


def _make_deg_call(w):
    return pl.kernel(
        functools.partial(_deg_body, w),
        out_type=jax.ShapeDtypeStruct((NC, N_PAD, w), _f32),
        mesh=_mesh,
        scratch_types=[
            pltpu.VMEM((K, CH), jnp.int32),
            pltpu.VMEM((CH, w), _f32),
            pltpu.VMEM_SHARED((N_PAD, w), _f32),
        ],
    )


_deg_call = _make_deg_call(DEG_W)


def _seg_body(h_hbm, src_hbm, dst_hbm, out_hbm, srcv, dstv, rows0, rows1,
              acc, gsem0, gsem1, ssem0, ssem1):
    c = lax.axis_index("c")
    s = lax.axis_index("s")
    wid = s * NC + c

    zero = jnp.zeros((16,), _f32)

    @pl.loop(0, CH)
    def _(r):
        @pl.loop(0, D, step=16)
        def _(cc):
            rows0[r, pl.ds(cc, 16)] = zero

    @pl.loop(0, BLK_PER_SUB)
    def _(b):
        off = (s * BLK_PER_SUB + b) * CH
        pltpu.sync_copy(rows0, acc.at[pl.ds(off, CH)])

    plsc.subcore_barrier()

    @pl.loop(0, 2)
    def _(hf):
        pltpu.sync_copy(src_hbm.at[wid].at[hf], srcv)
        pltpu.sync_copy(dst_hbm.at[wid].at[hf], dstv)
        pltpu.async_copy(h_hbm.at[srcv.at[0]], rows0, gsem0)

        @pl.loop(0, KH // 2)
        def _(t):
            j0 = 2 * t
            pltpu.make_async_copy(h_hbm.at[srcv.at[j0]], rows0, gsem0).wait()

            @pl.when(t > 0)
            def _():
                pltpu.make_async_copy(rows1, acc.at[dstv.at[j0]], ssem1).wait()

            pltpu.async_copy(h_hbm.at[srcv.at[j0 + 1]], rows1, gsem1)
            pltpu.async_copy(rows0, acc.at[dstv.at[j0]], ssem0, add=True)
            pltpu.make_async_copy(h_hbm.at[srcv.at[j0 + 1]], rows1, gsem1).wait()
            pltpu.make_async_copy(rows0, acc.at[dstv.at[j0]], ssem0).wait()

            @pl.when(t < KH // 2 - 1)
            def _():
                pltpu.async_copy(h_hbm.at[srcv.at[j0 + 2]], rows0, gsem0)

            pltpu.async_copy(rows1, acc.at[dstv.at[j0 + 1]], ssem1, add=True)

        pltpu.make_async_copy(rows1, acc.at[dstv.at[KH - 1]], ssem1).wait()

    plsc.subcore_barrier()

    @pl.loop(0, BLK_PER_SUB)
    def _(b):
        off = (s * BLK_PER_SUB + b) * CH
        pltpu.sync_copy(acc.at[pl.ds(off, CH)], out_hbm.at[c].at[pl.ds(off, CH)])


_seg_call = pl.kernel(
    _seg_body,
    out_type=jax.ShapeDtypeStruct((NC, N_PAD, D), _f32),
    mesh=_mesh,
    scratch_types=[
        pltpu.VMEM((KH, CH), jnp.int32),
        pltpu.VMEM((KH, CH), jnp.int32),
        pltpu.VMEM((CH, D), _f32),
        pltpu.VMEM((CH, D), _f32),
        pltpu.VMEM_SHARED((N_PAD, D), _f32),
        pltpu.SemaphoreType.DMA,
        pltpu.SemaphoreType.DMA,
        pltpu.SemaphoreType.DMA,
        pltpu.SemaphoreType.DMA,
    ],
)


B = 256


def _dinv_block(degp_ref):
    deg = degp_ref[0] + degp_ref[1] + 1.0
    return lax.rsqrt(deg)[:, 0:1]


def _row_mask(i):
    row = lax.broadcasted_iota(jnp.int32, (B, 1), 0) + i * B
    return row < N


def _tc1_body(x_ref, w_ref, degp_ref, out_ref):
    dinv = _dinv_block(degp_ref)
    h = jnp.dot(x_ref[...], w_ref[...], preferred_element_type=_f32)
    out_ref[...] = jnp.where(_row_mask(pl.program_id(0)), h * dinv, 0.0)


_tc1_call = pl.pallas_call(
    _tc1_body,
    grid=(N_PAD // B,),
    in_specs=[
        pl.BlockSpec((B, D), lambda i: (i, 0)),
        pl.BlockSpec((D, D), lambda i: (0, 0)),
        pl.BlockSpec((NC, B, DEG_W), lambda i: (0, i, 0)),
    ],
    out_specs=pl.BlockSpec((B, D), lambda i: (i, 0)),
    out_shape=jax.ShapeDtypeStruct((N_PAD, D), _f32),
)


def _tc2_body(s1_ref, h1_ref, degp_ref, w_ref, b1_ref, out_ref):
    dinv = _dinv_block(degp_ref)
    t = (s1_ref[0] + s1_ref[1] + h1_ref[...]) * dinv + b1_ref[...]
    z = jnp.where(t >= 0, t, 0.01 * t)
    h2 = jnp.dot(z, w_ref[...], preferred_element_type=_f32) * dinv
    out_ref[...] = jnp.where(_row_mask(pl.program_id(0)), h2, 0.0)


_tc2_call = pl.pallas_call(
    _tc2_body,
    grid=(N_PAD // B,),
    in_specs=[
        pl.BlockSpec((NC, B, D), lambda i: (0, i, 0)),
        pl.BlockSpec((B, D), lambda i: (i, 0)),
        pl.BlockSpec((NC, B, DEG_W), lambda i: (0, i, 0)),
        pl.BlockSpec((D, D), lambda i: (0, 0)),
        pl.BlockSpec((1, D), lambda i: (0, 0)),
    ],
    out_specs=pl.BlockSpec((B, D), lambda i: (i, 0)),
    out_shape=jax.ShapeDtypeStruct((N_PAD, D), _f32),
)


def _tc3_body(s2_ref, h2_ref, degp_ref, b2_ref, out_ref):
    dinv = _dinv_block(degp_ref)
    out_ref[...] = (s2_ref[0] + s2_ref[1] + h2_ref[...]) * dinv + b2_ref[...]


_tc3_call = pl.pallas_call(
    _tc3_body,
    grid=(N_PAD // B,),
    in_specs=[
        pl.BlockSpec((NC, B, D), lambda i: (0, i, 0)),
        pl.BlockSpec((B, D), lambda i: (i, 0)),
        pl.BlockSpec((NC, B, DEG_W), lambda i: (0, i, 0)),
        pl.BlockSpec((1, D), lambda i: (0, 0)),
    ],
    out_specs=pl.BlockSpec((B, D), lambda i: (i, 0)),
    out_shape=jax.ShapeDtypeStruct((N_PAD, D), _f32),
)


@jax.jit
def kernel(x, edge_index, W1, b1, W2, b2):
    src = edge_index[0].astype(jnp.int32)
    dst = edge_index[1].astype(jnp.int32)
    pad = PAD_ROW + (jnp.arange(E_PAD - E, dtype=jnp.int32) % (N_PAD - N))
    src_f = jnp.concatenate([src, pad])
    dst_f = jnp.concatenate([dst, pad])
    dst_p = dst_f.reshape(NW, K, CH)
    x_pad = jnp.concatenate([x, jnp.zeros((N_PAD - N, D), x.dtype)])

    src_h = src_f.reshape(NW, 2, KH, CH)
    dst_h = dst_f.reshape(NW, 2, KH, CH)

    degp = _deg_call(dst_p)
    h1p = _tc1_call(x_pad, W1, degp)
    s1p = _seg_call(h1p, src_h, dst_h)
    h2p = _tc2_call(s1p, h1p, degp, W2, b1.reshape(1, D))
    s2p = _seg_call(h2p, src_h, dst_h)
    out = _tc3_call(s2p, h2p, degp, b2.reshape(1, D))
    return out[:N]


# speedup vs baseline: 2.4865x; 1.0032x over previous
import functools

import jax
import jax.numpy as jnp
from jax import lax
from jax.experimental import pallas as pl
from jax.experimental.pallas import tpu as pltpu
from jax.experimental.pallas import tpu_sc as plsc

N = 10000
E = 320000
D = 128

NC = 2
NS = 16
NW = NC * NS
CH = 128
K = 80
KH = K // 2
E_PAD = NW * K * CH
N_PAD = 10240
BLK_PER_SUB = (N_PAD // CH) // NS
PAD_ROW = N
DEG_W = 128

_mesh = plsc.VectorSubcoreMesh(core_axis_name="c", subcore_axis_name="s")
_f32 = jnp.float32


def _deg_body(dst_hbm, out_hbm, dstv, buf, accd):
    c = lax.axis_index("c")
    s = lax.axis_index("s")
    wid = s * NC + c
    pltpu.sync_copy(dst_hbm.at[wid], dstv)

    zero = jnp.zeros((16,), _f32)
    one = jnp.ones((16,), _f32)

    @pl.loop(0, CH)
    def _(r):
        @pl.loop(0, DEG_W, step=16)
        def _(cc):
            buf[r, pl.ds(cc, 16)] = zero

    @pl.loop(0, BLK_PER_SUB)
    def _(b):
        off = (s * BLK_PER_SUB + b) * CH
        pltpu.sync_copy(buf, accd.at[pl.ds(off, CH)])

    @pl.loop(0, CH)
    def _(r):
        buf[r, pl.ds(0, 16)] = one

    plsc.subcore_barrier()

    @pl.loop(0, K)
    def _(j):
        pltpu.sync_copy(buf, accd.at[dstv.at[j]], add=True)

    plsc.subcore_barrier()

    @pl.loop(0, BLK_PER_SUB)
    def _(b):
        off = (s * BLK_PER_SUB + b) * CH
        pltpu.sync_copy(accd.at[pl.ds(off, CH)], out_hbm.at[c].at[pl.ds(off, CH)])


_deg_call = pl.kernel(
    _deg_body,
    out_type=jax.ShapeDtypeStruct((NC, N_PAD, DEG_W), _f32),
    mesh=_mesh,
    scratch_types=[
        pltpu.VMEM((K, CH), jnp.int32),
        pltpu.VMEM((CH, DEG_W), _f32),
        pltpu.VMEM_SHARED((N_PAD, DEG_W), _f32),
    ],
)


def _seg_body(h_hbm, src_hbm, dst_hbm, out_hbm, srcv, dstv, rows0, rows1,
              acc, gsem0, gsem1, ssem0, ssem1):
    c = lax.axis_index("c")
    s = lax.axis_index("s")
    wid = s * NC + c

    zero = jnp.zeros((16,), _f32)

    @pl.loop(0, CH)
    def _(r):
        @pl.loop(0, D, step=16)
        def _(cc):
            rows0[r, pl.ds(cc, 16)] = zero

    @pl.loop(0, BLK_PER_SUB)
    def _(b):
        off = (s * BLK_PER_SUB + b) * CH
        pltpu.sync_copy(rows0, acc.at[pl.ds(off, CH)])

    plsc.subcore_barrier()

    @pl.loop(0, 2)
    def _(hf):
        pltpu.sync_copy(src_hbm.at[wid].at[hf], srcv)
        pltpu.sync_copy(dst_hbm.at[wid].at[hf], dstv)
        pltpu.async_copy(h_hbm.at[srcv.at[0]], rows0, gsem0)

        @pl.loop(0, KH // 2)
        def _(t):
            j0 = 2 * t
            pltpu.make_async_copy(h_hbm.at[srcv.at[j0]], rows0, gsem0).wait()

            @pl.when(t > 0)
            def _():
                pltpu.make_async_copy(rows1, acc.at[dstv.at[j0]], ssem1).wait()

            pltpu.async_copy(h_hbm.at[srcv.at[j0 + 1]], rows1, gsem1)
            pltpu.async_copy(rows0, acc.at[dstv.at[j0]], ssem0, add=True)
            pltpu.make_async_copy(h_hbm.at[srcv.at[j0 + 1]], rows1, gsem1).wait()
            pltpu.make_async_copy(rows0, acc.at[dstv.at[j0]], ssem0).wait()

            @pl.when(t < KH // 2 - 1)
            def _():
                pltpu.async_copy(h_hbm.at[srcv.at[j0 + 2]], rows0, gsem0)

            pltpu.async_copy(rows1, acc.at[dstv.at[j0 + 1]], ssem1, add=True)

        pltpu.make_async_copy(rows1, acc.at[dstv.at[KH - 1]], ssem1).wait()

    plsc.subcore_barrier()

    @pl.loop(0, BLK_PER_SUB)
    def _(b):
        off = (s * BLK_PER_SUB + b) * CH
        pltpu.sync_copy(acc.at[pl.ds(off, CH)], out_hbm.at[c].at[pl.ds(off, CH)])


_seg_call = pl.kernel(
    _seg_body,
    out_type=jax.ShapeDtypeStruct((NC, N_PAD, D), _f32),
    mesh=_mesh,
    scratch_types=[
        pltpu.VMEM((KH, CH), jnp.int32),
        pltpu.VMEM((KH, CH), jnp.int32),
        pltpu.VMEM((CH, D), _f32),
        pltpu.VMEM((CH, D), _f32),
        pltpu.VMEM_SHARED((N_PAD, D), _f32),
        pltpu.SemaphoreType.DMA,
        pltpu.SemaphoreType.DMA,
        pltpu.SemaphoreType.DMA,
        pltpu.SemaphoreType.DMA,
    ],
)


B = 256
NBLK = N_PAD // B


def _dinv_block(degp_ref):
    deg = degp_ref[0] + degp_ref[1] + 1.0
    return lax.rsqrt(deg)[:, 0:1]


def _row_mask(i):
    row = lax.broadcasted_iota(jnp.int32, (B, 1), 0) + i * B
    return row < N


def _tc1_body(x_ref, w_ref, degp_ref, out_ref):
    dinv = _dinv_block(degp_ref)
    h = jnp.dot(x_ref[...], w_ref[...], preferred_element_type=_f32)
    out_ref[...] = jnp.where(_row_mask(pl.program_id(0)), h * dinv, 0.0)


_tc1_call = pl.pallas_call(
    _tc1_body,
    grid=(N_PAD // B,),
    in_specs=[
        pl.BlockSpec((B, D), lambda i: (i, 0)),
        pl.BlockSpec((D, D), lambda i: (0, 0)),
        pl.BlockSpec((NC, B, DEG_W), lambda i: (0, i, 0)),
    ],
    out_specs=pl.BlockSpec((B, D), lambda i: (i, 0)),
    out_shape=jax.ShapeDtypeStruct((N_PAD, D), _f32),
)


def _tc2_body(s1_ref, h1_ref, degp_ref, w_ref, b1_ref, out_ref):
    dinv = _dinv_block(degp_ref)
    t = (s1_ref[0] + s1_ref[1] + h1_ref[...]) * dinv + b1_ref[...]
    z = jnp.where(t >= 0, t, 0.01 * t)
    h2 = jnp.dot(z, w_ref[...], preferred_element_type=_f32) * dinv
    out_ref[...] = jnp.where(_row_mask(pl.program_id(0)), h2, 0.0)


_tc2_call = pl.pallas_call(
    _tc2_body,
    grid=(N_PAD // B,),
    in_specs=[
        pl.BlockSpec((NC, B, D), lambda i: (0, i, 0)),
        pl.BlockSpec((B, D), lambda i: (i, 0)),
        pl.BlockSpec((NC, B, DEG_W), lambda i: (0, i, 0)),
        pl.BlockSpec((D, D), lambda i: (0, 0)),
        pl.BlockSpec((1, D), lambda i: (0, 0)),
    ],
    out_specs=pl.BlockSpec((B, D), lambda i: (i, 0)),
    out_shape=jax.ShapeDtypeStruct((N_PAD, D), _f32),
)


def _tc3_body(s2_ref, h2_ref, degp_ref, b2_ref, out_ref):
    dinv = _dinv_block(degp_ref)
    out_ref[...] = (s2_ref[0] + s2_ref[1] + h2_ref[...]) * dinv + b2_ref[...]


_tc3_call = pl.pallas_call(
    _tc3_body,
    grid=(N_PAD // B,),
    in_specs=[
        pl.BlockSpec((NC, B, D), lambda i: (0, i, 0)),
        pl.BlockSpec((B, D), lambda i: (i, 0)),
        pl.BlockSpec((NC, B, DEG_W), lambda i: (0, i, 0)),
        pl.BlockSpec((1, D), lambda i: (0, 0)),
    ],
    out_specs=pl.BlockSpec((B, D), lambda i: (i, 0)),
    out_shape=jax.ShapeDtypeStruct((N_PAD, D), _f32),
)


@jax.jit
def kernel(x, edge_index, W1, b1, W2, b2):
    src = edge_index[0].astype(jnp.int32)
    dst = edge_index[1].astype(jnp.int32)
    pad = PAD_ROW + (jnp.arange(E_PAD - E, dtype=jnp.int32) % (N_PAD - N))
    src_f = jnp.concatenate([src, pad])
    dst_f = jnp.concatenate([dst, pad])
    dst_p = dst_f.reshape(NW, K, CH)
    x_pad = jnp.concatenate([x, jnp.zeros((N_PAD - N, D), x.dtype)])

    src_h = src_f.reshape(NW, 2, KH, CH)
    dst_h = dst_f.reshape(NW, 2, KH, CH)

    degp = _deg_call(dst_p)
    h1p = _tc1_call(x_pad, W1, degp)
    s1p = _seg_call(h1p, src_h, dst_h)
    h2p = _tc2_call(s1p, h1p, degp, W2, b1.reshape(1, D))
    s2p = _seg_call(h2p, src_h, dst_h)
    out = _tc3_call(s2p, h2p, degp, b2.reshape(1, D))
    return out[:N]
